# baseline (speedup 1.0000x reference)
import functools

import jax
import jax.numpy as jnp
from jax.experimental import pallas as pl
from jax.experimental.pallas import tpu as pltpu
from jax.experimental.pallas import tpu_sc as plsc

R_BLK = 512
LANES = 128
GATHER_WIN = 128


def _argmax_body(x_ref, tok_ref):
    blk = x_ref[0]
    c = blk.shape[1]
    n_chunks = c // LANES

    runmax = blk[:, 0:LANES]
    runcol = jnp.zeros((R_BLK, LANES), jnp.int32)
    for j in range(1, n_chunks):
        chunk = blk[:, j * LANES:(j + 1) * LANES]
        upd = chunk > runmax
        runmax = jnp.where(upd, chunk, runmax)
        runcol = jnp.where(upd, j, runcol)

    rowmax = jnp.max(runmax, axis=1, keepdims=True)
    lane = jax.lax.broadcasted_iota(jnp.int32, (R_BLK, LANES), 1)
    cfull = runcol * LANES + lane
    masked = jnp.where(runmax == rowmax, cfull, c)
    tok_ref[...] = jnp.min(masked, axis=1).reshape(1, R_BLK)


def _argmax_tokens(probs):
    b, c, h, w = probs.shape
    hw = h * w
    pt = probs.transpose(0, 2, 3, 1).reshape(b, hw, c)
    n_rb = hw // R_BLK

    out = pl.pallas_call(
        _argmax_body,
        grid=(b, n_rb),
        in_specs=[pl.BlockSpec((1, R_BLK, c), lambda i, r: (i, r, 0))],
        out_specs=pl.BlockSpec((1, R_BLK), lambda i, r: (0, i * n_rb + r)),
        out_shape=jax.ShapeDtypeStruct((1, b * hw), jnp.int32),
        compiler_params=pltpu.CompilerParams(
            dimension_semantics=("parallel", "arbitrary")
        ),
    )(pt)
    return out


def _sc_gather(embeddings, idx_flat):
    n = idx_flat.shape[1]
    d = embeddings.shape[1]
    mesh = plsc.VectorSubcoreMesh(core_axis_name="core", subcore_axis_name="subcore")

    @pl.kernel(
        out_type=jax.ShapeDtypeStruct((n, d), embeddings.dtype),
        mesh=mesh,
    )
    def gk(e_hbm, i_hbm, o_hbm):
        def body(i_vmem, o_vmem):
            pltpu.sync_copy(e_hbm.at[i_vmem.at[0]], o_vmem)

        pltpu.emit_pipeline(
            body,
            grid=(n // GATHER_WIN,),
            in_specs=[pl.BlockSpec((1, GATHER_WIN), lambda i: (0, i))],
            out_specs=[pl.BlockSpec((GATHER_WIN, d), lambda i: (i, 0))],
            core_axis_name=("core", "subcore"),
            dimension_semantics=(pltpu.PARALLEL,),
        )(i_hbm, o_hbm)

    return gk(embeddings, idx_flat)


def kernel(probs, tokens_shift, embeddings):
    b, c, h, w = probs.shape
    hw = h * w
    d = embeddings.shape[1]

    tokens = _argmax_tokens(probs) + tokens_shift
    idx = jnp.clip(tokens, 0, embeddings.shape[0] - 1)
    g = _sc_gather(embeddings, idx)
    x = g.reshape(b, h, w, d).transpose(0, 3, 1, 2)
    return x, tokens.reshape(b, h, w)


# speedup vs baseline: 1.0093x; 1.0093x over previous
import functools

import jax
import jax.numpy as jnp
from jax.experimental import pallas as pl
from jax.experimental.pallas import tpu as pltpu
from jax.experimental.pallas import tpu_sc as plsc

R_BLK = 256
LANES = 128
GATHER_WIN = 128


def _argmax_body(xa_ref, xb_ref, tok_ref):
    c = xa_ref.shape[2] * 2
    half_chunks = xa_ref.shape[2] // LANES

    runmax = xa_ref[0, :, 0:LANES]
    runcol = jnp.zeros((R_BLK, LANES), jnp.int32)
    for j in range(1, 2 * half_chunks):
        ref = xa_ref if j < half_chunks else xb_ref
        jj = j if j < half_chunks else j - half_chunks
        chunk = ref[0, :, jj * LANES:(jj + 1) * LANES]
        upd = chunk > runmax
        runmax = jnp.where(upd, chunk, runmax)
        runcol = jnp.where(upd, j, runcol)

    rowmax = jnp.max(runmax, axis=1, keepdims=True)
    lane = jax.lax.broadcasted_iota(jnp.int32, (R_BLK, LANES), 1)
    cfull = runcol * LANES + lane
    masked = jnp.where(runmax == rowmax, cfull, c)
    tok_ref[...] = jnp.min(masked, axis=1).reshape(1, R_BLK)


def _argmax_tokens(probs):
    b, c, h, w = probs.shape
    hw = h * w
    pt = probs.transpose(0, 2, 3, 1).reshape(b, hw, c)
    n_rb = hw // R_BLK

    out = pl.pallas_call(
        _argmax_body,
        grid=(b, n_rb),
        in_specs=[
            pl.BlockSpec((1, R_BLK, c // 2), lambda i, r: (i, r, 0)),
            pl.BlockSpec((1, R_BLK, c // 2), lambda i, r: (i, r, 1)),
        ],
        out_specs=pl.BlockSpec((1, R_BLK), lambda i, r: (0, i * n_rb + r)),
        out_shape=jax.ShapeDtypeStruct((1, b * hw), jnp.int32),
        compiler_params=pltpu.CompilerParams(
            dimension_semantics=("parallel", "arbitrary")
        ),
    )(pt, pt)
    return out


def _sc_gather(embeddings, idx_flat):
    n = idx_flat.shape[1]
    d = embeddings.shape[1]
    mesh = plsc.VectorSubcoreMesh(core_axis_name="core", subcore_axis_name="subcore")

    @pl.kernel(
        out_type=jax.ShapeDtypeStruct((n, d), embeddings.dtype),
        mesh=mesh,
    )
    def gk(e_hbm, i_hbm, o_hbm):
        def body(i_vmem, o_vmem):
            pltpu.sync_copy(e_hbm.at[i_vmem.at[0]], o_vmem)

        pltpu.emit_pipeline(
            body,
            grid=(n // GATHER_WIN,),
            in_specs=[pl.BlockSpec((1, GATHER_WIN), lambda i: (0, i))],
            out_specs=[pl.BlockSpec((GATHER_WIN, d), lambda i: (i, 0))],
            core_axis_name=("core", "subcore"),
            dimension_semantics=(pltpu.PARALLEL,),
        )(i_hbm, o_hbm)

    return gk(embeddings, idx_flat)


def kernel(probs, tokens_shift, embeddings):
    b, c, h, w = probs.shape
    hw = h * w
    d = embeddings.shape[1]

    tokens = _argmax_tokens(probs) + tokens_shift
    idx = jnp.clip(tokens, 0, embeddings.shape[0] - 1)
    g = _sc_gather(embeddings, idx)
    x = g.reshape(b, h, w, d).transpose(0, 3, 1, 2)
    return x, tokens.reshape(b, h, w)


# speedup vs baseline: 1.0178x; 1.0085x over previous
import functools

import jax
import jax.numpy as jnp
from jax.experimental import pallas as pl
from jax.experimental.pallas import tpu as pltpu
from jax.experimental.pallas import tpu_sc as plsc

R_BLK = 256
LANES = 128
GATHER_WIN = 128


def _argmax_body(shift_ref, x_ref, tok_ref, idx_ref):
    c = x_ref.shape[2]
    n_chunks = c // LANES

    runmax = x_ref[0, :, 0:LANES]
    runcol = jnp.zeros((R_BLK, LANES), jnp.int32)
    for j in range(1, n_chunks):
        chunk = x_ref[0, :, j * LANES:(j + 1) * LANES]
        upd = chunk > runmax
        runmax = jnp.where(upd, chunk, runmax)
        runcol = jnp.where(upd, j, runcol)

    rowmax = jnp.max(runmax, axis=1, keepdims=True)
    lane = jax.lax.broadcasted_iota(jnp.int32, (R_BLK, LANES), 1)
    cfull = runcol * LANES + lane
    masked = jnp.where(runmax == rowmax, cfull, c)
    amax = jnp.min(masked, axis=1).reshape(1, R_BLK)
    tokens = amax + shift_ref[0]
    tok_ref[...] = tokens
    idx_ref[...] = jnp.clip(tokens, 0, c - 1)


def _argmax_tokens(probs, tokens_shift):
    b, c, h, w = probs.shape
    hw = h * w
    pt = probs.transpose(0, 2, 3, 1).reshape(b, hw, c)
    n_rb = hw // R_BLK
    shift = jnp.asarray(tokens_shift, jnp.int32).reshape(1)

    tok, idx = pl.pallas_call(
        _argmax_body,
        grid=(b, n_rb),
        in_specs=[
            pl.BlockSpec(memory_space=pltpu.SMEM),
            pl.BlockSpec((1, R_BLK, c), lambda i, r: (i, r, 0)),
        ],
        out_specs=[
            pl.BlockSpec((1, R_BLK), lambda i, r: (0, i * n_rb + r)),
            pl.BlockSpec((1, R_BLK), lambda i, r: (0, i * n_rb + r)),
        ],
        out_shape=[
            jax.ShapeDtypeStruct((1, b * hw), jnp.int32),
            jax.ShapeDtypeStruct((1, b * hw), jnp.int32),
        ],
        compiler_params=pltpu.CompilerParams(
            dimension_semantics=("parallel", "arbitrary")
        ),
    )(shift, pt)
    return tok, idx


def _sc_gather(embeddings, idx_flat):
    n = idx_flat.shape[1]
    d = embeddings.shape[1]
    mesh = plsc.VectorSubcoreMesh(core_axis_name="core", subcore_axis_name="subcore")

    @pl.kernel(
        out_type=jax.ShapeDtypeStruct((n, d), embeddings.dtype),
        mesh=mesh,
    )
    def gk(e_hbm, i_hbm, o_hbm):
        def body(i_vmem, o_vmem):
            pltpu.sync_copy(e_hbm.at[i_vmem.at[0]], o_vmem)

        pltpu.emit_pipeline(
            body,
            grid=(n // GATHER_WIN,),
            in_specs=[pl.BlockSpec((1, GATHER_WIN), lambda i: (0, i))],
            out_specs=[pl.BlockSpec((GATHER_WIN, d), lambda i: (i, 0))],
            core_axis_name=("core", "subcore"),
            dimension_semantics=(pltpu.PARALLEL,),
        )(i_hbm, o_hbm)

    return gk(embeddings, idx_flat)


def kernel(probs, tokens_shift, embeddings):
    b, c, h, w = probs.shape
    hw = h * w
    d = embeddings.shape[1]

    tokens, idx = _argmax_tokens(probs, tokens_shift)
    g = _sc_gather(embeddings, idx)
    x = g.reshape(b, h, w, d).transpose(0, 3, 1, 2)
    return x, tokens.reshape(b, h, w)
